# baseline (speedup 1.0000x reference)
import functools

import jax
import jax.numpy as jnp
from jax import lax
from jax.experimental import pallas as pl
from jax.experimental.pallas import tpu as pltpu
from jax.experimental.pallas import tpu_sc as plsc

B = 16384
D = 32
NC = 2
NS = 16
NW = NC * NS
BPW = B // NW
CH = 128
NCHUNK = BPW // CH


def _sc_gather(user_idx, item_idx, embed_user_w, embed_item_w):
    mesh = plsc.VectorSubcoreMesh(core_axis_name="c", subcore_axis_name="s")

    @functools.partial(
        pl.kernel,
        mesh=mesh,
        compiler_params=pltpu.CompilerParams(use_tc_tiling_on_sc=False),
        out_type=(
            jax.ShapeDtypeStruct((B, D), jnp.float32),
            jax.ShapeDtypeStruct((B, D), jnp.float32),
        ),
        scratch_types=[
            pltpu.VMEM((NCHUNK, CH), jnp.int32),
            pltpu.VMEM((NCHUNK, CH), jnp.int32),
            pltpu.VMEM((BPW, D), jnp.float32),
            pltpu.VMEM((BPW, D), jnp.float32),
            pltpu.SemaphoreType.DMA,
            pltpu.SemaphoreType.DMA,
        ],
    )
    def k(uw_hbm, iw_hbm, uidx_hbm, iidx_hbm, eu_hbm, ei_hbm,
          uidx_v, iidx_v, urows_v, irows_v, sem_u, sem_i):
        wid = lax.axis_index("s") * NC + lax.axis_index("c")
        base = wid * BPW
        pltpu.sync_copy(uidx_hbm.at[wid], uidx_v)
        pltpu.sync_copy(iidx_hbm.at[wid], iidx_v)
        copies = []
        for j in range(NCHUNK):
            dst = pl.ds(j * CH, CH)
            copies.append(
                pltpu.async_copy(uw_hbm.at[uidx_v.at[j]], urows_v.at[dst], sem_u))
            copies.append(
                pltpu.async_copy(iw_hbm.at[iidx_v.at[j]], irows_v.at[dst], sem_i))
        for c in copies:
            c.wait()
        pltpu.sync_copy(urows_v, eu_hbm.at[pl.ds(base, BPW)])
        pltpu.sync_copy(irows_v, ei_hbm.at[pl.ds(base, BPW)])

    return k(embed_user_w, embed_item_w,
             user_idx.reshape(NW, NCHUNK, CH),
             item_idx.reshape(NW, NCHUNK, CH))


def _tc_loss_body(eu_ref, ei_ref, lab_ref, w_ref, b_ref, out_ref):
    t = eu_ref[...] * ei_ref[...] * w_ref[...]
    x = jnp.sum(t, axis=1) + b_ref[0]
    y = lab_ref[...]
    terms = jnp.maximum(x, 0.0) - x * y + jnp.log1p(jnp.exp(-jnp.abs(x)))
    out_ref[...] = (jnp.sum(terms) * (1.0 / B)).reshape(1, 1)


def _tc_loss(eu, ei, label, W, b):
    return pl.pallas_call(
        _tc_loss_body,
        out_shape=jax.ShapeDtypeStruct((1, 1), jnp.float32),
    )(eu, ei, label, W, b)


def kernel(user, item, label, embed_user_w, embed_item_w, W, b):
    eu, ei = _sc_gather(user, item, embed_user_w, embed_item_w)
    loss = _tc_loss(eu, ei, label, W, b)
    return loss.reshape(())
